# baseline (speedup 1.0000x reference)
import functools

import jax
import jax.numpy as jnp
from jax import lax
from jax.experimental import pallas as pl
from jax.experimental.pallas import tpu as pltpu
from jax.experimental.pallas import tpu_sc as plsc

N = 500000
NCAT = 1000
D = 64
H = 32

NC = 2
NS = 16
NW = NC * NS

NPAD = 512000
CPW = NPAD // NW
BS_SC = 1000
NIT = CPW // BS_SC

BT = 4000


def _prep_body(emb_ref, wcaT_ref, wcbT_ref, w2T_ref, b2_ref, bc_ref,
               t2_ref, m_ref):
    t2 = jnp.dot(emb_ref[...], wcaT_ref[...], preferred_element_type=jnp.float32)
    const = jnp.dot(b2_ref[...], wcbT_ref[...], preferred_element_type=jnp.float32)
    t2_ref[...] = t2 + const + bc_ref[...]
    m_ref[...] = jnp.dot(w2T_ref[...], wcbT_ref[...], preferred_element_type=jnp.float32)


def _sc_gather_body(idx_hbm, table_hbm, out_hbm, idx_v, rows_v, sem):
    wid = lax.axis_index("s") * NC + lax.axis_index("c")
    base = wid * CPW

    def body(j, carry):
        b = base + j * BS_SC
        pltpu.sync_copy(idx_hbm.at[pl.ds(b, BS_SC)], idx_v)
        pltpu.async_copy(table_hbm.at[idx_v], rows_v, sem).wait()
        pltpu.sync_copy(rows_v, out_hbm.at[pl.ds(b, BS_SC)])
        return carry

    lax.fori_loop(0, NIT, body, 0)


def _combine_body(g_ref, p_ref, c_ref, t_ref, e_ref, w1t_ref, b1_ref, m_ref,
                  o_ref):
    w1t = w1t_ref[...]
    h = (p_ref[...] * w1t[0:1, :] + c_ref[...] * w1t[1:2, :]
         + t_ref[...] * w1t[2:3, :] + e_ref[...] * w1t[3:4, :] + b1_ref[...])
    h = jnp.maximum(h, 0.0)
    o_ref[...] = g_ref[...] + jnp.dot(h, m_ref[...], preferred_element_type=jnp.float32)


def kernel(categories, popularity, cluster_ids, temporal_scores,
           geographic_scores, emb_table, W1, b1, W2, b2, Wc, bc):
    cats = categories.astype(jnp.int32)
    cats_pad = jnp.concatenate([cats, jnp.zeros((NPAD - N,), jnp.int32)])

    wcaT = Wc[:, :H].T
    wcbT = Wc[:, H:].T
    w2T = W2.T
    b2r = b2.reshape(1, H)
    bcr = bc.reshape(1, D)
    w1T = W1.T
    b1r = b1.reshape(1, H)

    t2, m = pl.pallas_call(
        _prep_body,
        out_shape=(
            jax.ShapeDtypeStruct((NCAT, D), jnp.float32),
            jax.ShapeDtypeStruct((H, D), jnp.float32),
        ),
    )(emb_table, wcaT, wcbT, w2T, b2r, bcr)

    mesh = plsc.VectorSubcoreMesh(core_axis_name="c", subcore_axis_name="s")
    gathered = pl.kernel(
        _sc_gather_body,
        mesh=mesh,
        out_type=jax.ShapeDtypeStruct((NPAD, D), jnp.float32),
        scratch_types=[
            pltpu.VMEM((BS_SC,), jnp.int32),
            pltpu.VMEM((BS_SC, D), jnp.float32),
            pltpu.SemaphoreType.DMA,
        ],
        compiler_params=pltpu.CompilerParams(use_tc_tiling_on_sc=False),
    )(cats_pad, t2)

    grid = N // BT
    out = pl.pallas_call(
        _combine_body,
        grid=(grid,),
        in_specs=[
            pl.BlockSpec((BT, D), lambda i: (i, 0)),
            pl.BlockSpec((BT, 1), lambda i: (i, 0)),
            pl.BlockSpec((BT, 1), lambda i: (i, 0)),
            pl.BlockSpec((BT, 1), lambda i: (i, 0)),
            pl.BlockSpec((BT, 1), lambda i: (i, 0)),
            pl.BlockSpec((4, H), lambda i: (0, 0)),
            pl.BlockSpec((1, H), lambda i: (0, 0)),
            pl.BlockSpec((H, D), lambda i: (0, 0)),
        ],
        out_specs=pl.BlockSpec((BT, D), lambda i: (i, 0)),
        out_shape=jax.ShapeDtypeStruct((N, D), jnp.float32),
    )(gathered, popularity.reshape(N, 1), cluster_ids.reshape(N, 1),
      temporal_scores.reshape(N, 1), geographic_scores.reshape(N, 1),
      w1T, b1r, m)
    return out


# speedup vs baseline: 1.0360x; 1.0360x over previous
import functools

import jax
import jax.numpy as jnp
from jax import lax
from jax.experimental import pallas as pl
from jax.experimental.pallas import tpu as pltpu
from jax.experimental.pallas import tpu_sc as plsc

N = 500000
NCAT = 1000
D = 64
H = 32

NC = 2
NS = 16
NW = NC * NS

NPAD = 512000
CPW = NPAD // NW
BS_SC = 1000
NIT = CPW // BS_SC
NB = 2

N2 = N // 2
BT2 = 2000


def _prep_body(emb_ref, wcaT_ref, wcbT_ref, w2T_ref, b2_ref, bc_ref,
               w1T_ref, b1_ref, t2_ref, w1d_ref, b1d_ref, md_ref):
    t2 = jnp.dot(emb_ref[...], wcaT_ref[...], preferred_element_type=jnp.float32)
    const = jnp.dot(b2_ref[...], wcbT_ref[...], preferred_element_type=jnp.float32)
    t2_ref[...] = t2 + const + bc_ref[...]
    m = jnp.dot(w2T_ref[...], wcbT_ref[...], preferred_element_type=jnp.float32)
    zm = jnp.zeros((H, D), jnp.float32)
    md_ref[...] = jnp.concatenate(
        [jnp.concatenate([m, zm], axis=1), jnp.concatenate([zm, m], axis=1)],
        axis=0)
    w1T = w1T_ref[...]
    zw = jnp.zeros((4, H), jnp.float32)
    w1d_ref[...] = jnp.concatenate(
        [jnp.concatenate([w1T, zw], axis=1), jnp.concatenate([zw, w1T], axis=1)],
        axis=0)
    b1 = b1_ref[...]
    b1d_ref[...] = jnp.concatenate([b1, b1], axis=1)


def _sc_gather_body(idx_hbm, table_hbm, out_hbm, idx_v, rows_v,
                    sem_i0, sem_i1, sem_g, sem_o0, sem_o1):
    wid = lax.axis_index("s") * NC + lax.axis_index("c")
    base = wid * CPW
    sem_i = [sem_i0, sem_i1]
    sem_o = [sem_o0, sem_o1]

    for b in range(NB):
        pltpu.async_copy(idx_hbm.at[pl.ds(base + b * BS_SC, BS_SC)],
                         idx_v.at[b], sem_i[b])

    def body(jo, carry):
        for b in range(NB):
            j = jo * NB + b
            off = base + j * BS_SC
            pltpu.make_async_copy(idx_hbm.at[pl.ds(off, BS_SC)],
                                  idx_v.at[b], sem_i[b]).wait()
            @pl.when(j >= NB)
            def _wait_prev():
                pltpu.make_async_copy(
                    rows_v.at[b],
                    out_hbm.at[pl.ds(off - NB * BS_SC, BS_SC)],
                    sem_o[b]).wait()
            pltpu.async_copy(table_hbm.at[idx_v.at[b]], rows_v.at[b], sem_g).wait()
            @pl.when(j + NB < NIT)
            def _prefetch():
                pltpu.async_copy(idx_hbm.at[pl.ds(off + NB * BS_SC, BS_SC)],
                                 idx_v.at[b], sem_i[b])
            pltpu.async_copy(rows_v.at[b], out_hbm.at[pl.ds(off, BS_SC)],
                             sem_o[b])
        return carry

    lax.fori_loop(0, NIT // NB, body, 0)

    for b in range(NB):
        off = base + (NIT - NB + b) * BS_SC
        pltpu.make_async_copy(rows_v.at[b], out_hbm.at[pl.ds(off, BS_SC)],
                              sem_o[b]).wait()


def _combine_body(g_ref, s_ref, w1d_ref, b1d_ref, md_ref, o_ref):
    h = jnp.dot(s_ref[...], w1d_ref[...], preferred_element_type=jnp.float32)
    h = jnp.maximum(h + b1d_ref[...], 0.0)
    o_ref[...] = g_ref[...] + jnp.dot(h, md_ref[...], preferred_element_type=jnp.float32)


def kernel(categories, popularity, cluster_ids, temporal_scores,
           geographic_scores, emb_table, W1, b1, W2, b2, Wc, bc):
    cats = categories.astype(jnp.int32)
    cats_pad = jnp.concatenate([cats, jnp.zeros((NPAD - N,), jnp.int32)])
    stats2 = jnp.stack([popularity, cluster_ids, temporal_scores,
                        geographic_scores], axis=-1).reshape(N2, 8)

    wcaT = Wc[:, :H].T
    wcbT = Wc[:, H:].T
    w2T = W2.T
    b2r = b2.reshape(1, H)
    bcr = bc.reshape(1, D)
    w1T = W1.T
    b1r = b1.reshape(1, H)

    t2, w1d, b1d, md = pl.pallas_call(
        _prep_body,
        out_shape=(
            jax.ShapeDtypeStruct((NCAT, D), jnp.float32),
            jax.ShapeDtypeStruct((8, D), jnp.float32),
            jax.ShapeDtypeStruct((1, D), jnp.float32),
            jax.ShapeDtypeStruct((D, 2 * D), jnp.float32),
        ),
    )(emb_table, wcaT, wcbT, w2T, b2r, bcr, w1T, b1r)

    mesh = plsc.VectorSubcoreMesh(core_axis_name="c", subcore_axis_name="s")
    gathered = pl.kernel(
        _sc_gather_body,
        mesh=mesh,
        out_type=jax.ShapeDtypeStruct((NPAD, D), jnp.float32),
        scratch_types=[
            pltpu.VMEM((NB, BS_SC), jnp.int32),
            pltpu.VMEM((NB, BS_SC, D), jnp.float32),
            pltpu.SemaphoreType.DMA,
            pltpu.SemaphoreType.DMA,
            pltpu.SemaphoreType.DMA,
            pltpu.SemaphoreType.DMA,
            pltpu.SemaphoreType.DMA,
        ],
        compiler_params=pltpu.CompilerParams(use_tc_tiling_on_sc=False),
    )(cats_pad, t2)
    g2 = gathered.reshape(NPAD // 2, 2 * D)

    grid = N2 // BT2
    out2 = pl.pallas_call(
        _combine_body,
        grid=(grid,),
        in_specs=[
            pl.BlockSpec((BT2, 2 * D), lambda i: (i, 0)),
            pl.BlockSpec((BT2, 8), lambda i: (i, 0)),
            pl.BlockSpec((8, D), lambda i: (0, 0)),
            pl.BlockSpec((1, D), lambda i: (0, 0)),
            pl.BlockSpec((D, 2 * D), lambda i: (0, 0)),
        ],
        out_specs=pl.BlockSpec((BT2, 2 * D), lambda i: (i, 0)),
        out_shape=jax.ShapeDtypeStruct((N2, 2 * D), jnp.float32),
    )(g2, stats2, w1d, b1d, md)
    return out2.reshape(N, D)
